# baseline (speedup 1.0000x reference)
import jax
import jax.numpy as jnp
import numpy as np
from jax import lax
from jax.experimental import pallas as pl
from jax.experimental.pallas import tpu as pltpu
from jax.experimental.pallas import tpu_sc as plsc

ISHAPE = (1024, 1024, 3)
POOL = (7, 7)
NLVL = 4
H = 128
W = 128
C = 256
NROI = 1000
NPAD = 1024
NWORK = 32
SLOTS = NPAD // NWORK
NPTS = POOL[0] * POOL[1]
GROWS = 56
LANES = 16
YSC = np.float32((H - 1.0) / (POOL[0] - 1.0))
XSC = np.float32((W - 1.0) / (POOL[1] - 1.0))
GCHUNK = POOL[1] * C


def _sc_body(table_h, aug_h, out_h,
             aug_v, crop2,
             iA0, iA1, iA2, iA3, iB0, iB1, iB2, iB3,
             rA0, rA1, rA2, rA3, rB0, rB1, rB2, rB3,
             gsemA, gsemB, osem):
    cid = lax.axis_index("c")
    sid = lax.axis_index("s")
    wid = sid * 2 + cid
    base = wid * SLOTS

    pltpu.sync_copy(aug_h.at[pl.ds(base, SLOTS)], aug_v)

    nvalid = jnp.minimum(jnp.int32(SLOTS), jnp.int32(NROI) - base)
    it = lax.iota(jnp.int32, LANES)

    idxA = (iA0, iA1, iA2, iA3)
    idxB = (iB0, iB1, iB2, iB3)
    rowsA = (rA0, rA1, rA2, rA3)
    rowsB = (rB0, rB1, rB2, rB3)

    def fire(k, idx4, r4, gsem):
        @pl.when(k < nvalid)
        def _():
            rec = aug_v[k]
            y1n = rec[0]
            x1n = rec[1]
            y2n = rec[2]
            x2n = rec[3]
            lbn = rec[4].astype(jnp.int32)
            for b in range(4):
                pv = it + (b * LANES)
                iv = lax.shift_right_logical(pv * 9363, 16)
                jv = pv - iv * POOL[1]
                ys = y1n * (H - 1.0) + iv.astype(jnp.float32) * (y2n - y1n) * YSC
                xs = x1n * (W - 1.0) + jv.astype(jnp.float32) * (x2n - x1n) * XSC
                ysc = jnp.minimum(jnp.maximum(ys, 0.0), H - 1.0)
                xsc = jnp.minimum(jnp.maximum(xs, 0.0), W - 1.0)
                y0 = ysc.astype(jnp.int32)
                x0 = xsc.astype(jnp.int32)
                y1i = jnp.minimum(y0 + 1, H - 1)
                x1i = jnp.minimum(x0 + 1, W - 1)
                sl = pl.ds(b * LANES, LANES)
                row0 = lbn + y0 * W
                row1 = lbn + y1i * W
                idx4[0][sl] = row0 + x0
                idx4[1][sl] = row0 + x1i
                idx4[2][sl] = row1 + x0
                idx4[3][sl] = row1 + x1i
            for q in range(4):
                pltpu.async_copy(table_h.at[idx4[q].at[pl.ds(0, GROWS)]],
                                 r4[q], gsem)

    def process(k, idx4, r4, gsem):
        @pl.when(k < nvalid)
        def _():
            for q in range(4):
                pltpu.make_async_copy(table_h.at[idx4[q].at[pl.ds(0, GROWS)]],
                                      r4[q], gsem).wait()
            rec = aug_v[k]
            y1n = rec[0]
            x1n = rec[1]
            y2n = rec[2]
            x2n = rec[3]

            def gbody(g, cy):
                par = g & 1

                @pl.when(g >= 2)
                def _():
                    pltpu.make_async_copy(
                        crop2.at[0], out_h.at[base, pl.ds(0, GCHUNK)],
                        osem).wait()

                for r in range(POOL[1]):
                    p = g * POOL[1] + r
                    ii = lax.shift_right_logical(p * 9363, 16)
                    jj = p - ii * POOL[1]
                    ysp = y1n * (H - 1.0) + ii.astype(jnp.float32) * (y2n - y1n) * YSC
                    xsp = x1n * (W - 1.0) + jj.astype(jnp.float32) * (x2n - x1n) * XSC
                    ysc = jnp.minimum(jnp.maximum(ysp, 0.0), H - 1.0)
                    xsc = jnp.minimum(jnp.maximum(xsp, 0.0), W - 1.0)
                    fy = ysc.astype(jnp.int32).astype(jnp.float32)
                    fy = fy - jnp.where(fy > ysc, 1.0, 0.0)
                    fx = xsc.astype(jnp.int32).astype(jnp.float32)
                    fx = fx - jnp.where(fx > xsc, 1.0, 0.0)
                    wyv = jnp.full((LANES,), ysc - fy)
                    wxv = jnp.full((LANES,), xsc - fx)
                    for cc in range(C // LANES):
                        cs = pl.ds(cc * LANES, LANES)
                        tl = r4[0][p, cs]
                        tr = r4[1][p, cs]
                        bl = r4[2][p, cs]
                        br = r4[3][p, cs]
                        top = tl + (tr - tl) * wxv
                        bot = bl + (br - bl) * wxv
                        crop2[par, pl.ds(r * C + cc * LANES, LANES)] = (
                            top + (bot - top) * wyv)

                go = pl.multiple_of(g * GCHUNK, GCHUNK)
                pltpu.async_copy(crop2.at[par],
                                 out_h.at[base + k, pl.ds(go, GCHUNK)], osem)
                return cy

            lax.fori_loop(0, POOL[0], gbody, 0)
            for _ in range(2):
                pltpu.make_async_copy(
                    crop2.at[0], out_h.at[base, pl.ds(0, GCHUNK)], osem).wait()

    fire(jnp.int32(0), idxA, rowsA, gsemA)

    def iter_body(i, cy):
        k0 = i * 2
        k1 = k0 + 1
        fire(k1, idxB, rowsB, gsemB)
        process(k0, idxA, rowsA, gsemA)
        fire(k0 + 2, idxA, rowsA, gsemA)
        process(k1, idxB, rowsB, gsemB)
        return cy

    lax.fori_loop(0, SLOTS // 2, iter_body, 0)


@jax.jit
def _run(table, aug):
    mesh = plsc.VectorSubcoreMesh(core_axis_name="c", subcore_axis_name="s",
                                  num_cores=2, num_subcores=16)
    f = pl.kernel(
        _sc_body,
        out_type=jax.ShapeDtypeStruct((NROI, NPTS * C), jnp.float32),
        mesh=mesh,
        scratch_types=[
            pltpu.VMEM((SLOTS, LANES), jnp.float32),
            pltpu.VMEM((2, GCHUNK), jnp.float32),
            pltpu.VMEM((64,), jnp.int32),
            pltpu.VMEM((64,), jnp.int32),
            pltpu.VMEM((64,), jnp.int32),
            pltpu.VMEM((64,), jnp.int32),
            pltpu.VMEM((64,), jnp.int32),
            pltpu.VMEM((64,), jnp.int32),
            pltpu.VMEM((64,), jnp.int32),
            pltpu.VMEM((64,), jnp.int32),
            pltpu.VMEM((GROWS, C), jnp.float32),
            pltpu.VMEM((GROWS, C), jnp.float32),
            pltpu.VMEM((GROWS, C), jnp.float32),
            pltpu.VMEM((GROWS, C), jnp.float32),
            pltpu.VMEM((GROWS, C), jnp.float32),
            pltpu.VMEM((GROWS, C), jnp.float32),
            pltpu.VMEM((GROWS, C), jnp.float32),
            pltpu.VMEM((GROWS, C), jnp.float32),
            pltpu.SemaphoreType.DMA,
            pltpu.SemaphoreType.DMA,
            pltpu.SemaphoreType.DMA,
        ],
    )
    return f(table, aug)


def kernel(feature_maps, rois):
    roi = rois[0]
    y1, x1, y2, x2 = jnp.split(roi, 4, axis=1)
    h = y2 - y1
    w = x2 - x1
    lvl = jnp.log(jnp.sqrt(h * w)) / jnp.log(2.0)
    lvl = jnp.minimum(3, jnp.maximum(0, jnp.round(lvl - 5.0).astype(jnp.int32)))
    lvl = jnp.squeeze(lvl, axis=1)
    order = jnp.argsort(lvl)

    scale = jnp.array([1.0 / ISHAPE[0], 1.0 / ISHAPE[1],
                       1.0 / ISHAPE[0], 1.0 / ISHAPE[1]], dtype=jnp.float32)
    norm = roi * scale
    rec = jnp.concatenate(
        [norm[order], (lvl[order] * (H * W)).astype(jnp.float32)[:, None],
         jnp.zeros((NROI, LANES - 5), jnp.float32)], axis=1)
    aug = jnp.zeros((NPAD, LANES), jnp.float32).at[:NROI].set(rec)

    table = feature_maps[:, 0].reshape(NLVL * H * W, C)
    out = _run(table, aug)
    return out.reshape(1, NROI, POOL[0], POOL[1], C)


# speedup vs baseline: 1.0754x; 1.0754x over previous
import jax
import jax.numpy as jnp
import numpy as np
from jax import lax
from jax.experimental import pallas as pl
from jax.experimental.pallas import tpu as pltpu
from jax.experimental.pallas import tpu_sc as plsc

ISHAPE = (1024, 1024, 3)
POOL = (7, 7)
NLVL = 4
H = 128
W = 128
C = 256
NROI = 1000
NPAD = 1024
NWORK = 32
SLOTS = NPAD // NWORK
NPTS = POOL[0] * POOL[1]
GROWS = 56
LANES = 16
YSC = np.float32((H - 1.0) / (POOL[0] - 1.0))
XSC = np.float32((W - 1.0) / (POOL[1] - 1.0))
GCHUNK = POOL[1] * C


def _sc_body(table_h, aug_h, out_h,
             aug_v, crop,
             idxA, idxB,
             rA0, rA1, rA2, rA3, rB0, rB1, rB2, rB3,
             gsemA, gsemB):
    cid = lax.axis_index("c")
    sid = lax.axis_index("s")
    wid = sid * 2 + cid
    base = wid * SLOTS

    pltpu.sync_copy(aug_h.at[pl.ds(base * LANES, SLOTS * LANES)], aug_v)

    nvalid = jnp.minimum(jnp.int32(SLOTS), jnp.int32(NROI) - base)
    it = lax.iota(jnp.int32, LANES)

    rowsA = (rA0, rA1, rA2, rA3)
    rowsB = (rB0, rB1, rB2, rB3)

    def fire(k, idxS, r4, gsem):
        @pl.when(k < nvalid)
        def _():
            rec = aug_v[pl.ds(pl.multiple_of(k * LANES, LANES), LANES)]
            y1n = rec[0]
            x1n = rec[1]
            y2n = rec[2]
            x2n = rec[3]
            lbn = rec[4].astype(jnp.int32)
            for b in range(4):
                pv = it + (b * LANES)
                iv = lax.shift_right_logical(pv * 9363, 16)
                jv = pv - iv * POOL[1]
                ys = y1n * (H - 1.0) + iv.astype(jnp.float32) * (y2n - y1n) * YSC
                xs = x1n * (W - 1.0) + jv.astype(jnp.float32) * (x2n - x1n) * XSC
                ysc = jnp.minimum(jnp.maximum(ys, 0.0), H - 1.0)
                xsc = jnp.minimum(jnp.maximum(xs, 0.0), W - 1.0)
                y0 = ysc.astype(jnp.int32)
                x0 = xsc.astype(jnp.int32)
                y1i = jnp.minimum(y0 + 1, H - 1)
                x1i = jnp.minimum(x0 + 1, W - 1)
                row0 = lbn + y0 * W
                row1 = lbn + y1i * W
                idxS[pl.ds(b * LANES, LANES)] = row0 + x0
                idxS[pl.ds(64 + b * LANES, LANES)] = row0 + x1i
                idxS[pl.ds(128 + b * LANES, LANES)] = row1 + x0
                idxS[pl.ds(192 + b * LANES, LANES)] = row1 + x1i
            for q in range(4):
                pltpu.async_copy(table_h.at[idxS.at[pl.ds(64 * q, GROWS)]],
                                 r4[q], gsem)

    def process(k, idxS, r4, gsem):
        @pl.when(k < nvalid)
        def _():
            for q in range(4):
                pltpu.make_async_copy(table_h.at[idxS.at[pl.ds(64 * q, GROWS)]],
                                      r4[q], gsem).wait()
            rec = aug_v[pl.ds(pl.multiple_of(k * LANES, LANES), LANES)]
            y1n = rec[0]
            x1n = rec[1]
            y2n = rec[2]
            x2n = rec[3]

            def pbody(p, cy):
                    ii = lax.shift_right_logical(p * 9363, 16)
                    jj = p - ii * POOL[1]
                    ysp = y1n * (H - 1.0) + ii.astype(jnp.float32) * (y2n - y1n) * YSC
                    xsp = x1n * (W - 1.0) + jj.astype(jnp.float32) * (x2n - x1n) * XSC
                    ysc = jnp.minimum(jnp.maximum(ysp, 0.0), H - 1.0)
                    xsc = jnp.minimum(jnp.maximum(xsp, 0.0), W - 1.0)
                    fy = ysc.astype(jnp.int32).astype(jnp.float32)
                    fy = fy - jnp.where(fy > ysc, 1.0, 0.0)
                    fx = xsc.astype(jnp.int32).astype(jnp.float32)
                    fx = fx - jnp.where(fx > xsc, 1.0, 0.0)
                    wyv = jnp.full((LANES,), ysc - fy)
                    wxv = jnp.full((LANES,), xsc - fx)
                    pc = pl.multiple_of(p * C, C)
                    for cc in range(C // LANES):
                        cs = pl.ds(cc * LANES, LANES)
                        tl = r4[0][p, cs]
                        tr = r4[1][p, cs]
                        bl = r4[2][p, cs]
                        br = r4[3][p, cs]
                        top = tl + (tr - tl) * wxv
                        bot = bl + (br - bl) * wxv
                        crop[pl.ds(pc + cc * LANES, LANES)] = (
                            top + (bot - top) * wyv)
                    return cy

            lax.fori_loop(0, NPTS, pbody, 0)
            pltpu.sync_copy(crop, out_h.at[base + k])

    fire(jnp.int32(0), idxA, rowsA, gsemA)

    def iter_body(i, cy):
        k0 = i * 2
        k1 = k0 + 1
        fire(k1, idxB, rowsB, gsemB)
        process(k0, idxA, rowsA, gsemA)
        fire(k0 + 2, idxA, rowsA, gsemA)
        process(k1, idxB, rowsB, gsemB)
        return cy

    lax.fori_loop(0, SLOTS // 2, iter_body, 0)


@jax.jit
def _run(table, aug):
    mesh = plsc.VectorSubcoreMesh(core_axis_name="c", subcore_axis_name="s",
                                  num_cores=2, num_subcores=16)
    f = pl.kernel(
        _sc_body,
        out_type=jax.ShapeDtypeStruct((NROI, NPTS * C), jnp.float32),
        mesh=mesh,
        scratch_types=[
            pltpu.VMEM((SLOTS * LANES,), jnp.float32),
            pltpu.VMEM((NPTS * C,), jnp.float32),
            pltpu.VMEM((256,), jnp.int32),
            pltpu.VMEM((256,), jnp.int32),
            pltpu.VMEM((GROWS, C), jnp.float32),
            pltpu.VMEM((GROWS, C), jnp.float32),
            pltpu.VMEM((GROWS, C), jnp.float32),
            pltpu.VMEM((GROWS, C), jnp.float32),
            pltpu.VMEM((GROWS, C), jnp.float32),
            pltpu.VMEM((GROWS, C), jnp.float32),
            pltpu.VMEM((GROWS, C), jnp.float32),
            pltpu.VMEM((GROWS, C), jnp.float32),
            pltpu.SemaphoreType.DMA,
            pltpu.SemaphoreType.DMA,
        ],
    )
    return f(table, aug)


def kernel(feature_maps, rois):
    roi = rois[0]
    y1, x1, y2, x2 = jnp.split(roi, 4, axis=1)
    h = y2 - y1
    w = x2 - x1
    lvl = jnp.log(jnp.sqrt(h * w)) / jnp.log(2.0)
    lvl = jnp.minimum(3, jnp.maximum(0, jnp.round(lvl - 5.0).astype(jnp.int32)))
    lvl = jnp.squeeze(lvl, axis=1)
    order = jnp.argsort(lvl)

    scale = jnp.array([1.0 / ISHAPE[0], 1.0 / ISHAPE[1],
                       1.0 / ISHAPE[0], 1.0 / ISHAPE[1]], dtype=jnp.float32)
    norm = roi * scale
    rec = jnp.concatenate(
        [norm[order], (lvl[order] * (H * W)).astype(jnp.float32)[:, None],
         jnp.zeros((NROI, LANES - 5), jnp.float32)], axis=1)
    aug = jnp.zeros((NPAD, LANES), jnp.float32).at[:NROI].set(rec).reshape(-1)

    table = feature_maps[:, 0].reshape(NLVL * H * W, C)
    out = _run(table, aug)
    return out.reshape(1, NROI, POOL[0], POOL[1], C)


# speedup vs baseline: 2.0742x; 1.9287x over previous
import functools

import jax
import jax.numpy as jnp
import numpy as np
from jax import lax
from jax.experimental import pallas as pl
from jax.experimental.pallas import tpu as pltpu
from jax.experimental.pallas import tpu_sc as plsc

ISHAPE = (1024, 1024, 3)
POOL = (7, 7)
NLVL = 4
H = 128
W = 128
C = 256
NROI = 1000
NPAD = 1024
NWORK = 32
SLOTS = NPAD // NWORK
NPTS = POOL[0] * POOL[1]
GROWS = 56
LANES = 16
YSC = np.float32((H - 1.0) / (POOL[0] - 1.0))
XSC = np.float32((W - 1.0) / (POOL[1] - 1.0))


def _sc_body(table_h, aug_h, order_h, out_h,
             order_v, aug_v,
             idx_tl, idx_tr, idx_bl, idx_br,
             r_tl, r_tr, r_bl, r_br, crop, gsem):
    cid = lax.axis_index("c")
    sid = lax.axis_index("s")
    wid = sid * 2 + cid
    base = wid * SLOTS

    pltpu.sync_copy(order_h.at[pl.ds(base, SLOTS)], order_v)
    pltpu.async_copy(aug_h.at[order_v], aug_v, gsem).wait()

    nvalid = jnp.minimum(jnp.int32(SLOTS), jnp.int32(NROI) - base)
    it = lax.iota(jnp.int32, LANES)

    def roi_body(k, carry):
        @pl.when(k < nvalid)
        def _():
            rec = aug_v[k, pl.ds(0, LANES)]
            y1n = rec[0]
            x1n = rec[1]
            y2n = rec[2]
            x2n = rec[3]
            lbn = rec[4].astype(jnp.int32)

            for b in range(4):
                pv = it + (b * LANES)
                iv = lax.shift_right_logical(pv * 9363, 16)
                jv = pv - iv * POOL[1]
                ivf = iv.astype(jnp.float32)
                jvf = jv.astype(jnp.float32)
                ys = y1n * (H - 1.0) + ivf * (y2n - y1n) * YSC
                xs = x1n * (W - 1.0) + jvf * (x2n - x1n) * XSC
                vy = jnp.where((ys >= 0.0) & (ys <= H - 1.0), 1.0, 0.0)
                vx = jnp.where((xs >= 0.0) & (xs <= W - 1.0), 1.0, 0.0)
                ysc = jnp.minimum(jnp.maximum(ys, 0.0), H - 1.0)
                xsc = jnp.minimum(jnp.maximum(xs, 0.0), W - 1.0)
                y0 = ysc.astype(jnp.int32)
                x0 = xsc.astype(jnp.int32)
                ly = ysc - y0.astype(jnp.float32)
                lx = xsc - x0.astype(jnp.float32)
                y1i = jnp.minimum(y0 + 1, H - 1)
                x1i = jnp.minimum(x0 + 1, W - 1)
                sl = pl.ds(b * LANES, LANES)
                row0 = lbn + y0 * W
                row1 = lbn + y1i * W
                idx_tl[sl] = row0 + x0
                idx_tr[sl] = row0 + x1i
                idx_bl[sl] = row1 + x0
                idx_br[sl] = row1 + x1i

            c1 = pltpu.async_copy(table_h.at[idx_tl.at[pl.ds(0, GROWS)]], r_tl, gsem)
            c2 = pltpu.async_copy(table_h.at[idx_tr.at[pl.ds(0, GROWS)]], r_tr, gsem)
            c3 = pltpu.async_copy(table_h.at[idx_bl.at[pl.ds(0, GROWS)]], r_bl, gsem)
            c4 = pltpu.async_copy(table_h.at[idx_br.at[pl.ds(0, GROWS)]], r_br, gsem)
            c1.wait()
            c2.wait()
            c3.wait()
            c4.wait()

            def pbody(p, cy):
                ii = lax.shift_right_logical(p * 9363, 16)
                jj = p - ii * POOL[1]
                ysp = y1n * (H - 1.0) + ii.astype(jnp.float32) * (y2n - y1n) * YSC
                xsp = x1n * (W - 1.0) + jj.astype(jnp.float32) * (x2n - x1n) * XSC
                ysc = jnp.minimum(jnp.maximum(ysp, 0.0), H - 1.0)
                xsc = jnp.minimum(jnp.maximum(xsp, 0.0), W - 1.0)
                fy = ysc.astype(jnp.int32).astype(jnp.float32)
                fy = fy - jnp.where(fy > ysc, 1.0, 0.0)
                fx = xsc.astype(jnp.int32).astype(jnp.float32)
                fx = fx - jnp.where(fx > xsc, 1.0, 0.0)
                wy = ysc - fy
                wx = xsc - fx
                wyv = jnp.full((LANES,), wy)
                wxv = jnp.full((LANES,), wx)
                for cc in range(C // LANES):
                    cs = pl.ds(cc * LANES, LANES)
                    tl = r_tl[p, cs]
                    tr = r_tr[p, cs]
                    bl = r_bl[p, cs]
                    br = r_br[p, cs]
                    top = tl + (tr - tl) * wxv
                    bot = bl + (br - bl) * wxv
                    crop[p, cs] = top + (bot - top) * wyv
                return cy

            lax.fori_loop(0, NPTS, pbody, 0)
            pltpu.sync_copy(crop, out_h.at[base + k])
        return carry

    lax.fori_loop(0, SLOTS, roi_body, 0)


@jax.jit
def _run(table, aug, order_pad):
    mesh = plsc.VectorSubcoreMesh(core_axis_name="c", subcore_axis_name="s",
                                  num_cores=2, num_subcores=16)
    f = pl.kernel(
        _sc_body,
        out_type=jax.ShapeDtypeStruct((NROI, NPTS, C), jnp.float32),
        mesh=mesh,
        scratch_types=[
            pltpu.VMEM((SLOTS,), jnp.int32),
            pltpu.VMEM((SLOTS, 128), jnp.float32),
            pltpu.VMEM((64,), jnp.int32),
            pltpu.VMEM((64,), jnp.int32),
            pltpu.VMEM((64,), jnp.int32),
            pltpu.VMEM((64,), jnp.int32),
            pltpu.VMEM((GROWS, C), jnp.float32),
            pltpu.VMEM((GROWS, C), jnp.float32),
            pltpu.VMEM((GROWS, C), jnp.float32),
            pltpu.VMEM((GROWS, C), jnp.float32),
            pltpu.VMEM((NPTS, C), jnp.float32),
            pltpu.SemaphoreType.DMA,
        ],
    )
    return f(table, aug, order_pad)


def kernel(feature_maps, rois):
    roi = rois[0]
    y1, x1, y2, x2 = jnp.split(roi, 4, axis=1)
    h = y2 - y1
    w = x2 - x1
    lvl = jnp.log(jnp.sqrt(h * w)) / jnp.log(2.0)
    lvl = jnp.minimum(3, jnp.maximum(0, jnp.round(lvl - 5.0).astype(jnp.int32)))
    lvl = jnp.squeeze(lvl, axis=1)
    order = jnp.argsort(lvl)

    scale = jnp.array([1.0 / ISHAPE[0], 1.0 / ISHAPE[1],
                       1.0 / ISHAPE[0], 1.0 / ISHAPE[1]], dtype=jnp.float32)
    norm = roi * scale
    rec = jnp.concatenate(
        [norm, (lvl * (H * W)).astype(jnp.float32)[:, None],
         jnp.zeros((NROI, 123), jnp.float32)], axis=1)
    aug = jnp.zeros((NPAD, 128), jnp.float32).at[:NROI].set(rec)
    order_pad = jnp.concatenate(
        [order.astype(jnp.int32), jnp.zeros((NPAD - NROI,), jnp.int32)])

    table = feature_maps[:, 0].reshape(NLVL * H * W, C)
    out = _run(table, aug, order_pad)
    return out.reshape(1, NROI, POOL[0], POOL[1], C)
